# baseline (speedup 1.0000x reference)
import functools

import jax
import jax.numpy as jnp
from jax import lax
from jax.experimental import pallas as pl
from jax.experimental.pallas import tpu as pltpu
from jax.experimental.pallas import tpu_sc as plsc

NUM_TASKS = 3
PROMPT_LEN = 20
HIDDEN = 4096
BATCH = 1024

NUM_CORES = 2
NUM_SUBCORES = 16
NUM_WORKERS = NUM_CORES * NUM_SUBCORES

ROWS = BATCH * PROMPT_LEN
ROWS_PER_WORKER = ROWS // NUM_WORKERS
CHUNK = 8
N_CHUNKS = ROWS_PER_WORKER // CHUNK


def _sc_gather(row_idx, table):
    mesh = plsc.VectorSubcoreMesh(core_axis_name="c", subcore_axis_name="s")

    @functools.partial(
        pl.kernel,
        out_type=jax.ShapeDtypeStruct((ROWS, HIDDEN), jnp.float32),
        mesh=mesh,
        scratch_types=[
            pltpu.VMEM((ROWS_PER_WORKER,), jnp.int32),
            pltpu.VMEM((CHUNK, HIDDEN), jnp.float32),
            pltpu.VMEM((CHUNK, HIDDEN), jnp.float32),
            pltpu.SemaphoreType.DMA,
            pltpu.SemaphoreType.DMA,
            pltpu.SemaphoreType.DMA,
            pltpu.SemaphoreType.DMA,
        ],
    )
    def run(idx_hbm, table_hbm, out_hbm, idx_v, buf0, buf1, gsem0, gsem1,
            ssem0, ssem1):
        wid = lax.axis_index("s") * NUM_CORES + lax.axis_index("c")
        base = wid * ROWS_PER_WORKER
        pltpu.sync_copy(idx_hbm.at[pl.ds(base, ROWS_PER_WORKER)], idx_v)

        def gather(c, buf, sem):
            return pltpu.async_copy(
                table_hbm.at[idx_v.at[pl.ds(c * CHUNK, CHUNK)]], buf, sem)

        def gather_wait(buf, sem):
            pltpu.make_async_copy(
                table_hbm.at[idx_v.at[pl.ds(0, CHUNK)]], buf, sem).wait()

        def scatter(c, buf, sem):
            return pltpu.async_copy(
                buf, out_hbm.at[pl.ds(base + c * CHUNK, CHUNK)], sem)

        def scatter_wait(buf, sem):
            pltpu.make_async_copy(
                buf, out_hbm.at[pl.ds(base, CHUNK)], sem).wait()

        gather(0, buf0, gsem0)
        gather(1, buf1, gsem1)

        last = N_CHUNKS - 1

        @pl.loop(0, N_CHUNKS, step=2)
        def _(g):
            gather_wait(buf0, gsem0)
            scatter(g, buf0, ssem0)
            gather_wait(buf1, gsem1)
            scatter(g + 1, buf1, ssem1)
            scatter_wait(buf0, ssem0)
            gather(jnp.minimum(g + 2, last), buf0, gsem0)
            scatter_wait(buf1, ssem1)
            gather(jnp.minimum(g + 3, last), buf1, gsem1)

        gather_wait(buf0, gsem0)
        gather_wait(buf1, gsem1)

    return run(row_idx, table)


def kernel(task_ids, prompt_embeddings):
    row_idx = (task_ids.astype(jnp.int32)[:, None] * PROMPT_LEN
               + jnp.arange(PROMPT_LEN, dtype=jnp.int32)).reshape(ROWS)
    table = prompt_embeddings.reshape(NUM_TASKS * PROMPT_LEN, HIDDEN)
    out = _sc_gather(row_idx, table)
    return out.reshape(BATCH, PROMPT_LEN, HIDDEN)
